# baseline (speedup 1.0000x reference)
import functools
import math

import jax
import jax.numpy as jnp
from jax import lax
from jax.experimental import pallas as pl
from jax.experimental.pallas import tpu as pltpu
from jax.experimental.pallas import tpu_sc as plsc

_L = 16
_NW = 32
_LB = 128
_TW = 128


def _transpose_body(v128, d, tableT_hbm, tail_hbm, out_hbm,
                    gbufs, sbufs, tail_v, gsems, ssems):
    scale = jnp.float32(math.sqrt(d))
    bw = 2 * _TW
    nblk = v128 // bw
    na = -(-nblk // _NW)
    na = na + (na & 1)
    wid = lax.axis_index("s") * 2 + lax.axis_index("c")
    iota = lax.iota(jnp.int32, _L)

    def blk_of(i):
        return jnp.minimum(i * _NW + wid, nblk - 1)

    def start_gather(i, b):
        off = blk_of(i) * bw
        pltpu.async_copy(
            tableT_hbm.at[pl.ds(0, d), pl.ds(off, bw)], gbufs[b], gsems[b])

    def wait_gather(b):
        pltpu.make_async_copy(
            tableT_hbm.at[pl.ds(0, d), pl.ds(0, bw)], gbufs[b],
            gsems[b]).wait()

    def transpose(b):
        gbuf, sbuf = gbufs[b], sbufs[b]
        kf = d // _L

        @plsc.parallel_loop(0, bw * kf, unroll=4)
        def _(i):
            row = iota + (i & (kf - 1)) * _L
            col = iota * 0 + (i // kf)
            val = plsc.load_gather(gbuf, [row, col])
            sbuf[pl.ds(i * _L, _L)] = val * scale

    def start_write(i, b):
        off = pl.multiple_of(blk_of(i) * (bw * d), 8)
        pltpu.async_copy(
            sbufs[b], out_hbm.at[pl.ds(off, bw * d)], ssems[b])

    def wait_write(b):
        pltpu.make_async_copy(
            sbufs[b], out_hbm.at[pl.ds(0, bw * d)], ssems[b]).wait()

    @pl.when(wid == 0)
    def _():
        pltpu.sync_copy(tail_hbm, tail_v)
        pltpu.sync_copy(tail_v, out_hbm.at[pl.ds(v128 * d, tail_v.shape[0])])

    for b in range(2):
        start_gather(b, b)
    for g in range(2):
        b = g & 1
        wait_gather(b)
        transpose(b)
        start_gather(g + 2, b)
        start_write(g, b)

    def round_body(r, carry):
        g0 = r * 2
        for b in range(2):
            g = g0 + b
            wait_gather(b)
            wait_write(b)
            transpose(b)
            start_gather(g + 2, b)
            start_write(g, b)
        return carry

    lax.fori_loop(1, na // 2 - 1, round_body, 0)

    for b in range(2):
        g = na - 2 + b
        wait_gather(b)
        wait_write(b)
        transpose(b)
        start_write(g, b)
    for b in range(2):
        wait_write(b)


def _gather_body(nt, d, table_hbm, idx_hbm, out_hbm,
                 idx_v, cidxs, gbufs, sbufs, gsems, ssems):
    er_n = d // 8
    plane = d * _NW * _LB
    wid = lax.axis_index("s") * 2 + lax.axis_index("c")
    n_per_w = _LB * nt
    pltpu.sync_copy(idx_hbm.at[pl.ds(wid * n_per_w, n_per_w)], idx_v)

    iota = lax.iota(jnp.int32, _L)
    rows = [iota + 16 * m for m in range(8)]

    def prep(t, b):
        for m in range(8):
            pos = (iota + 16 * m) * nt + t
            cidxs[b][pl.ds(16 * m, _L)] = plsc.load_gather(idx_v, [pos])

    def start_gather(b):
        pltpu.async_copy(table_hbm.at[cidxs[b]], gbufs[b], gsems[b])

    def wait_gather(b):
        pltpu.make_async_copy(
            table_hbm.at[cidxs[b]], gbufs[b], gsems[b]).wait()

    def transpose(b):
        gbuf, sbuf = gbufs[b], sbufs[b]

        @plsc.parallel_loop(0, er_n * 64, unroll=4)
        def _(o):
            e = ((o // 64) * 8) + ((o // 8) & 7)
            row = iota + (o & 7) * _L
            col = iota * 0 + e
            val = plsc.load_gather(gbuf, [row, col])
            sbuf[o // 64, 0, pl.ds((o & 63) * _L, _L)] = val

    def start_writes(t, b):
        pltpu.async_copy(
            sbufs[b],
            out_hbm.at[pl.ds(t * er_n, er_n), pl.ds(wid, 1), pl.ds(0, 1024)],
            ssems[b])

    def drain_writes(b):
        pltpu.make_async_copy(
            out_hbm.at[pl.ds(0, er_n), pl.ds(0, 1), pl.ds(0, 1024)],
            sbufs[b], ssems[b]).wait()

    for b in range(2):
        prep(b, b)
        start_gather(b)
    for g in range(2):
        b = g & 1
        wait_gather(b)
        transpose(b)
        prep(g + 2, b)
        start_gather(b)
        start_writes(g, b)

    def round_body(r, carry):
        g0 = r * 2
        for b in range(2):
            g = g0 + b
            wait_gather(b)
            drain_writes(b)
            transpose(b)
            prep(g + 2, b)
            start_gather(b)
            start_writes(g, b)
        return carry

    lax.fori_loop(1, nt // 2 - 1, round_body, 0)

    for b in range(2):
        g = nt - 2 + b
        wait_gather(b)
        drain_writes(b)
        transpose(b)
        start_writes(g, b)
    for b in range(2):
        drain_writes(b)


def kernel(tokens, table):
    v, d = table.shape
    bt, nt = tokens.shape
    idx = tokens.reshape(-1).astype(jnp.int32)
    v128 = (v // _TW) * _TW
    scale = jnp.float32(math.sqrt(d))
    tail = (table[v128:] * scale).reshape(-1)
    mesh = plsc.VectorSubcoreMesh(core_axis_name="c", subcore_axis_name="s")

    fa = pl.kernel(
        functools.partial(_transpose_body, v128, d),
        mesh=mesh,
        compiler_params=pltpu.CompilerParams(
            use_tc_tiling_on_sc=True, needs_layout_passes=False),
        out_type=jax.ShapeDtypeStruct((v * d,), jnp.float32),
        scratch_types=[
            [pltpu.VMEM((d, 2 * _TW), jnp.float32) for _ in range(2)],
            [pltpu.VMEM((2 * _TW * d,), jnp.float32) for _ in range(2)],
            pltpu.VMEM(((v - v128) * d,), jnp.float32),
            [pltpu.SemaphoreType.DMA for _ in range(2)],
            [pltpu.SemaphoreType.DMA for _ in range(2)],
        ],
    )
    table_c = fa(table.T, tail).reshape(v, d)

    fb = pl.kernel(
        functools.partial(_gather_body, nt, d),
        mesh=mesh,
        compiler_params=pltpu.CompilerParams(
            use_tc_tiling_on_sc=False, needs_layout_passes=False),
        out_type=jax.ShapeDtypeStruct((nt * (d // 8), _NW, 1024), jnp.float32),
        scratch_types=[
            pltpu.VMEM((_LB * nt,), jnp.int32),
            [pltpu.VMEM((_LB,), jnp.int32) for _ in range(2)],
            [pltpu.VMEM((_LB, d), jnp.float32) for _ in range(2)],
            [pltpu.VMEM((d // 8, 1, 1024), jnp.float32) for _ in range(2)],
            [pltpu.SemaphoreType.DMA for _ in range(2)],
            [pltpu.SemaphoreType.DMA for _ in range(2)],
        ],
    )
    out1 = fb(table_c, idx)
    o = out1.reshape(nt, d // 8, _NW, 8, _TW)
    o = jnp.transpose(o, (2, 4, 0, 1, 3))
    return o.reshape(bt, nt, d)


# speedup vs baseline: 1.0017x; 1.0017x over previous
import functools
import math

import jax
import jax.numpy as jnp
from jax import lax
from jax.experimental import pallas as pl
from jax.experimental.pallas import tpu as pltpu
from jax.experimental.pallas import tpu_sc as plsc

_L = 16
_NW = 32
_LB = 128
_TW = 128


def _transpose_body(v128, d, tableT_hbm, tail_hbm, out_hbm,
                    gbufs, sbufs, tail_v, gsems, ssems):
    scale = jnp.float32(math.sqrt(d))
    bw = 2 * _TW
    nblk = v128 // bw
    base_n = nblk // _NW
    na = nblk - base_n * (_NW - 1)
    na = na + (na & 1)
    wid = lax.axis_index("s") * 2 + lax.axis_index("c")
    iota = lax.iota(jnp.int32, _L)

    def blk_of(i):
        return jnp.minimum(wid * base_n + i, nblk - 1)

    def start_gather(i, b):
        off = blk_of(i) * bw
        pltpu.async_copy(
            tableT_hbm.at[pl.ds(0, d), pl.ds(off, bw)], gbufs[b], gsems[b])

    def wait_gather(b):
        pltpu.make_async_copy(
            tableT_hbm.at[pl.ds(0, d), pl.ds(0, bw)], gbufs[b],
            gsems[b]).wait()

    def transpose(b):
        gbuf, sbuf = gbufs[b], sbufs[b]
        kf = d // _L

        @plsc.parallel_loop(0, bw * kf, unroll=4)
        def _(i):
            row = iota + (i & (kf - 1)) * _L
            col = iota * 0 + (i // kf)
            val = plsc.load_gather(gbuf, [row, col])
            sbuf[pl.ds(i * _L, _L)] = val * scale

    def start_write(i, b):
        off = pl.multiple_of(blk_of(i) * (bw * d), 8)
        pltpu.async_copy(
            sbufs[b], out_hbm.at[pl.ds(off, bw * d)], ssems[b])

    def wait_write(b):
        pltpu.make_async_copy(
            sbufs[b], out_hbm.at[pl.ds(0, bw * d)], ssems[b]).wait()

    @pl.when(wid == 0)
    def _():
        pltpu.sync_copy(tail_hbm, tail_v)
        pltpu.sync_copy(tail_v, out_hbm.at[pl.ds(v128 * d, tail_v.shape[0])])

    for b in range(2):
        start_gather(b, b)
    for g in range(2):
        b = g & 1
        wait_gather(b)
        transpose(b)
        start_gather(g + 2, b)
        start_write(g, b)

    def round_body(r, carry):
        g0 = r * 2
        for b in range(2):
            g = g0 + b
            wait_gather(b)
            wait_write(b)
            transpose(b)
            start_gather(g + 2, b)
            start_write(g, b)
        return carry

    lax.fori_loop(1, na // 2 - 1, round_body, 0)

    for b in range(2):
        g = na - 2 + b
        wait_gather(b)
        wait_write(b)
        transpose(b)
        start_write(g, b)
    for b in range(2):
        wait_write(b)


def _gather_body(nt, d, table_hbm, idx_hbm, out_hbm,
                 idx_v, cidxs, gbufs, sbufs, gsems, ssems):
    er_n = d // 8
    plane = d * _NW * _LB
    wid = lax.axis_index("s") * 2 + lax.axis_index("c")
    n_per_w = _LB * nt
    pltpu.sync_copy(idx_hbm.at[pl.ds(wid * n_per_w, n_per_w)], idx_v)

    iota = lax.iota(jnp.int32, _L)
    toff = (wid * nt) // _NW

    def prep(g, b):
        t = lax.rem(g + toff, nt)
        for m in range(8):
            pos = (iota + 16 * m) * nt + t
            cidxs[b][pl.ds(16 * m, _L)] = plsc.load_gather(idx_v, [pos])

    def start_gather(b):
        pltpu.async_copy(table_hbm.at[cidxs[b]], gbufs[b], gsems[b])

    def wait_gather(b):
        pltpu.make_async_copy(
            table_hbm.at[cidxs[b]], gbufs[b], gsems[b]).wait()

    def transpose(b):
        gbuf, sbuf = gbufs[b], sbufs[b]

        @plsc.parallel_loop(0, er_n * 64, unroll=4)
        def _(o):
            e = ((o // 64) * 8) + ((o // 8) & 7)
            row = iota + (o & 7) * _L
            col = iota * 0 + e
            val = plsc.load_gather(gbuf, [row, col])
            sbuf[o // 64, 0, pl.ds((o & 63) * _L, _L)] = val

    def start_writes(g, b):
        t = lax.rem(g + toff, nt)
        pltpu.async_copy(
            sbufs[b],
            out_hbm.at[pl.ds(t * er_n, er_n), pl.ds(wid, 1), pl.ds(0, 1024)],
            ssems[b])

    def drain_writes(b):
        pltpu.make_async_copy(
            out_hbm.at[pl.ds(0, er_n), pl.ds(0, 1), pl.ds(0, 1024)],
            sbufs[b], ssems[b]).wait()

    for b in range(2):
        prep(b, b)
        start_gather(b)
    for g in range(2):
        b = g & 1
        wait_gather(b)
        transpose(b)
        prep(g + 2, b)
        start_gather(b)
        start_writes(g, b)

    def round_body(r, carry):
        g0 = r * 2
        for b in range(2):
            g = g0 + b
            wait_gather(b)
            drain_writes(b)
            transpose(b)
            prep(g + 2, b)
            start_gather(b)
            start_writes(g, b)
        return carry

    lax.fori_loop(1, nt // 2 - 1, round_body, 0)

    for b in range(2):
        g = nt - 2 + b
        wait_gather(b)
        drain_writes(b)
        transpose(b)
        start_writes(g, b)
    for b in range(2):
        drain_writes(b)


def kernel(tokens, table):
    v, d = table.shape
    bt, nt = tokens.shape
    idx = tokens.reshape(-1).astype(jnp.int32)
    v128 = (v // _TW) * _TW
    scale = jnp.float32(math.sqrt(d))
    tail = (table[v128:] * scale).reshape(-1)
    mesh = plsc.VectorSubcoreMesh(core_axis_name="c", subcore_axis_name="s")

    fa = pl.kernel(
        functools.partial(_transpose_body, v128, d),
        mesh=mesh,
        compiler_params=pltpu.CompilerParams(
            use_tc_tiling_on_sc=True, needs_layout_passes=False),
        out_type=jax.ShapeDtypeStruct((v * d,), jnp.float32),
        scratch_types=[
            [pltpu.VMEM((d, 2 * _TW), jnp.float32) for _ in range(2)],
            [pltpu.VMEM((2 * _TW * d,), jnp.float32) for _ in range(2)],
            pltpu.VMEM(((v - v128) * d,), jnp.float32),
            [pltpu.SemaphoreType.DMA for _ in range(2)],
            [pltpu.SemaphoreType.DMA for _ in range(2)],
        ],
    )
    table_c = fa(table.T, tail).reshape(v, d)

    fb = pl.kernel(
        functools.partial(_gather_body, nt, d),
        mesh=mesh,
        compiler_params=pltpu.CompilerParams(
            use_tc_tiling_on_sc=False, needs_layout_passes=False),
        out_type=jax.ShapeDtypeStruct((nt * (d // 8), _NW, 1024), jnp.float32),
        scratch_types=[
            pltpu.VMEM((_LB * nt,), jnp.int32),
            [pltpu.VMEM((_LB,), jnp.int32) for _ in range(2)],
            [pltpu.VMEM((_LB, d), jnp.float32) for _ in range(2)],
            [pltpu.VMEM((d // 8, 1, 1024), jnp.float32) for _ in range(2)],
            [pltpu.SemaphoreType.DMA for _ in range(2)],
            [pltpu.SemaphoreType.DMA for _ in range(2)],
        ],
    )
    out1 = fb(table_c, idx)
    o = out1.reshape(nt, d // 8, _NW, 8, _TW)
    o = jnp.transpose(o, (2, 4, 0, 1, 3))
    return o.reshape(bt, nt, d)


# speedup vs baseline: 1.0037x; 1.0019x over previous
import functools
import math

import jax
import jax.numpy as jnp
from jax import lax
from jax.experimental import pallas as pl
from jax.experimental.pallas import tpu as pltpu
from jax.experimental.pallas import tpu_sc as plsc

_L = 16
_NW = 32
_LB = 128
_TW = 128


def _transpose_body(v128, d, tableT_hbm, tail_hbm, out_hbm,
                    gbufs, sbufs, tail_v, gsems, ssems):
    scale = jnp.float32(math.sqrt(d))
    bw = 2 * _TW
    nblk = v128 // bw
    base_n = nblk // _NW
    na = nblk - base_n * (_NW - 1)
    na = na + (na & 1)
    wid = lax.axis_index("s") * 2 + lax.axis_index("c")
    iota = lax.iota(jnp.int32, _L)

    def blk_of(i):
        return jnp.minimum(wid * base_n + i, nblk - 1)

    def start_gather(i, b):
        off = blk_of(i) * bw
        pltpu.async_copy(
            tableT_hbm.at[pl.ds(0, d), pl.ds(off, bw)], gbufs[b], gsems[b])

    def wait_gather(b):
        pltpu.make_async_copy(
            tableT_hbm.at[pl.ds(0, d), pl.ds(0, bw)], gbufs[b],
            gsems[b]).wait()

    def transpose(b):
        gbuf, sbuf = gbufs[b], sbufs[b]
        kf = d // _L

        @plsc.parallel_loop(0, bw * kf, unroll=4)
        def _(i):
            row = iota + (i & (kf - 1)) * _L
            col = iota * 0 + (i // kf)
            val = plsc.load_gather(gbuf, [row, col])
            sbuf[pl.ds(i * _L, _L)] = val * scale

    def start_write(i, b):
        off = pl.multiple_of(blk_of(i) * (bw * d), 8)
        pltpu.async_copy(
            sbufs[b], out_hbm.at[pl.ds(off, bw * d)], ssems[b])

    def wait_write(b):
        pltpu.make_async_copy(
            sbufs[b], out_hbm.at[pl.ds(0, bw * d)], ssems[b]).wait()

    @pl.when(wid == 0)
    def _():
        pltpu.sync_copy(tail_hbm, tail_v)
        pltpu.sync_copy(tail_v, out_hbm.at[pl.ds(v128 * d, tail_v.shape[0])])

    for b in range(2):
        start_gather(b, b)

    def round_body(r, carry):
        g0 = r * 2
        for b in range(2):
            g = g0 + b
            wait_gather(b)

            @pl.when(g >= 2)
            def _():
                wait_write(b)

            transpose(b)

            @pl.when(g + 2 < na)
            def _():
                start_gather(g + 2, b)

            start_write(g, b)
        return carry

    lax.fori_loop(0, na // 2, round_body, 0)

    for b in range(2):
        wait_write(b)


def _gather_body(nt, d, table_hbm, idx_hbm, out_hbm,
                 idx_v, cidxs, gbufs, sbufs, gsems, ssems):
    er_n = d // 8
    plane = d * _NW * _LB
    wid = lax.axis_index("s") * 2 + lax.axis_index("c")
    n_per_w = _LB * nt
    pltpu.sync_copy(idx_hbm.at[pl.ds(wid * n_per_w, n_per_w)], idx_v)

    iota = lax.iota(jnp.int32, _L)
    toff = (wid * nt) // _NW

    def prep(g, b):
        t = lax.rem(g + toff, nt)
        for m in range(8):
            pos = (iota + 16 * m) * nt + t
            cidxs[b][pl.ds(16 * m, _L)] = plsc.load_gather(idx_v, [pos])

    def start_gather(b):
        pltpu.async_copy(table_hbm.at[cidxs[b]], gbufs[b], gsems[b])

    def wait_gather(b):
        pltpu.make_async_copy(
            table_hbm.at[cidxs[b]], gbufs[b], gsems[b]).wait()

    def transpose(b):
        gbuf, sbuf = gbufs[b], sbufs[b]

        @plsc.parallel_loop(0, er_n * 64, unroll=4)
        def _(o):
            e = ((o // 64) * 8) + ((o // 8) & 7)
            row = iota + (o & 7) * _L
            col = iota * 0 + e
            val = plsc.load_gather(gbuf, [row, col])
            sbuf[o // 64, 0, pl.ds((o & 63) * _L, _L)] = val

    def start_writes(g, b):
        t = lax.rem(g + toff, nt)
        pltpu.async_copy(
            sbufs[b],
            out_hbm.at[pl.ds(t * er_n, er_n), pl.ds(wid, 1), pl.ds(0, 1024)],
            ssems[b])

    def drain_writes(b):
        pltpu.make_async_copy(
            out_hbm.at[pl.ds(0, er_n), pl.ds(0, 1), pl.ds(0, 1024)],
            sbufs[b], ssems[b]).wait()

    for b in range(2):
        prep(b, b)
        start_gather(b)

    def round_body(r, carry):
        g0 = r * 2
        for b in range(2):
            g = g0 + b
            wait_gather(b)

            @pl.when(g >= 2)
            def _():
                drain_writes(b)

            transpose(b)
            prep(g + 2, b)

            @pl.when(g + 2 < nt)
            def _():
                start_gather(b)

            start_writes(g, b)
        return carry

    lax.fori_loop(0, nt // 2, round_body, 0)

    for b in range(2):
        drain_writes(b)


def kernel(tokens, table):
    v, d = table.shape
    bt, nt = tokens.shape
    idx = tokens.reshape(-1).astype(jnp.int32)
    v128 = (v // _TW) * _TW
    scale = jnp.float32(math.sqrt(d))
    tail = (table[v128:] * scale).reshape(-1)
    mesh = plsc.VectorSubcoreMesh(core_axis_name="c", subcore_axis_name="s")

    fa = pl.kernel(
        functools.partial(_transpose_body, v128, d),
        mesh=mesh,
        compiler_params=pltpu.CompilerParams(
            use_tc_tiling_on_sc=True, needs_layout_passes=False),
        out_type=jax.ShapeDtypeStruct((v * d,), jnp.float32),
        scratch_types=[
            [pltpu.VMEM((d, 2 * _TW), jnp.float32) for _ in range(2)],
            [pltpu.VMEM((2 * _TW * d,), jnp.float32) for _ in range(2)],
            pltpu.VMEM(((v - v128) * d,), jnp.float32),
            [pltpu.SemaphoreType.DMA for _ in range(2)],
            [pltpu.SemaphoreType.DMA for _ in range(2)],
        ],
    )
    table_c = fa(table.T, tail).reshape(v, d)

    fb = pl.kernel(
        functools.partial(_gather_body, nt, d),
        mesh=mesh,
        compiler_params=pltpu.CompilerParams(
            use_tc_tiling_on_sc=False, needs_layout_passes=False),
        out_type=jax.ShapeDtypeStruct((nt * (d // 8), _NW, 1024), jnp.float32),
        scratch_types=[
            pltpu.VMEM((_LB * nt,), jnp.int32),
            [pltpu.VMEM((_LB,), jnp.int32) for _ in range(2)],
            [pltpu.VMEM((_LB, d), jnp.float32) for _ in range(2)],
            [pltpu.VMEM((d // 8, 1, 1024), jnp.float32) for _ in range(2)],
            [pltpu.SemaphoreType.DMA for _ in range(2)],
            [pltpu.SemaphoreType.DMA for _ in range(2)],
        ],
    )
    out1 = fb(table_c, idx)
    o = out1.reshape(nt, d // 8, _NW, 8, _TW)
    o = jnp.transpose(o, (2, 4, 0, 1, 3))
    return o.reshape(bt, nt, d)


# speedup vs baseline: 1.3441x; 1.3392x over previous
import functools
import math

import jax
import jax.numpy as jnp
from jax import lax
from jax.experimental import pallas as pl
from jax.experimental.pallas import tpu as pltpu
from jax.experimental.pallas import tpu_sc as plsc

_LANES = 16


def _emb_body(b_per_w, chunk, nchunks, d,
              table_hbm, idx_hbm, out_hbm,
              idx_v, gbufs, sbufs, gsems, ssems):
    scale = jnp.float32(math.sqrt(d))
    wid = lax.axis_index("s") * 2 + lax.axis_index("c")
    base = wid * b_per_w
    pltpu.sync_copy(idx_hbm.at[pl.ds(base, b_per_w)], idx_v)

    def start_gather(g, b):
        off = pl.multiple_of(g * chunk, chunk)
        pltpu.async_copy(
            table_hbm.at[idx_v.at[pl.ds(off, chunk)]], gbufs[b], gsems[b]
        )

    def wait_gather(b):
        pltpu.make_async_copy(
            table_hbm.at[idx_v.at[pl.ds(0, chunk)]], gbufs[b], gsems[b]
        ).wait()

    def start_write(g, b):
        off = pl.multiple_of(g * chunk, chunk)
        pltpu.async_copy(
            sbufs[b], out_hbm.at[pl.ds(base + off, chunk)], ssems[b]
        )

    def wait_write(b):
        pltpu.make_async_copy(
            sbufs[b], out_hbm.at[pl.ds(base, chunk)], ssems[b]
        ).wait()

    def do_scale(b):
        gbuf, sbuf = gbufs[b], sbufs[b]

        def row_body(i, c):
            for j in range(d // _LANES):
                sl = pl.ds(j * _LANES, _LANES)
                sbuf[i, sl] = gbuf[i, sl] * scale
            return c

        lax.fori_loop(0, chunk, row_body, 0, unroll=4)

    start_gather(0, 0)
    start_gather(1, 1)

    for b in range(2):
        wait_gather(b)
        do_scale(b)
        start_gather(2 + b, b)
        start_write(b, b)

    def round_body(r, carry):
        g0 = r * 2
        for b in range(2):
            g = g0 + b
            wait_gather(b)
            wait_write(b)
            do_scale(b)
            start_gather(g + 2, b)
            start_write(g, b)
        return carry

    lax.fori_loop(1, nchunks // 2 - 1, round_body, 0)

    for b in range(2):
        g = nchunks - 2 + b
        wait_gather(b)
        wait_write(b)
        do_scale(b)
        start_write(g, b)
    for b in range(2):
        wait_write(b)


def kernel(tokens, table):
    v, d = table.shape
    idx = tokens.reshape(-1).astype(jnp.int32)
    b = idx.shape[0]
    nw = 32
    b_per_w = b // nw
    chunk = 128
    nchunks = b_per_w // chunk

    mesh = plsc.VectorSubcoreMesh(core_axis_name="c", subcore_axis_name="s")
    f = pl.kernel(
        functools.partial(_emb_body, b_per_w, chunk, nchunks, d),
        mesh=mesh,
        compiler_params=pltpu.CompilerParams(use_tc_tiling_on_sc=False),
        out_type=jax.ShapeDtypeStruct((b, d), jnp.float32),
        scratch_types=[
            pltpu.VMEM((b_per_w,), jnp.int32),
            [pltpu.VMEM((chunk, d), jnp.float32) for _ in range(2)],
            [pltpu.VMEM((chunk, d), jnp.float32) for _ in range(2)],
            [pltpu.SemaphoreType.DMA for _ in range(2)],
            [pltpu.SemaphoreType.DMA for _ in range(2)],
        ],
    )
    out = f(table, idx)
    return out.reshape(*tokens.shape, d)
